# baseline (speedup 1.0000x reference)
import functools

import jax
import jax.numpy as jnp
from jax import lax
from jax.experimental import pallas as pl
from jax.experimental.pallas import tpu as pltpu
from jax.experimental.pallas import tpu_sc as plsc

N = 10000
E = 160000
D = 256
HALF = 128
NS = 16
NP = N + NS
CH = 128
EPS = E // NS
NCH = 80
PAD = NCH * CH - EPS
PH = 2
PCH = NCH // PH
RPS = NP // NS
RD = 80
RDS = RD // NS

_MESH = plsc.VectorSubcoreMesh(core_axis_name="c", subcore_axis_name="s")


@functools.partial(
    pl.kernel,
    out_type=(
        jax.ShapeDtypeStruct((NS, RD * 128), jnp.int32),
        jax.ShapeDtypeStruct((NS, RD * 128), jnp.int32),
    ),
    mesh=_MESH,
    scratch_types=[
        pltpu.VMEM((NCH, CH), jnp.int32),
        pltpu.VMEM((RD * 128,), jnp.int32),
    ],
    compiler_params=pltpu.CompilerParams(needs_layout_passes=False),
)
def _deg_kernel(src_hbm, dst_hbm, outdeg_hbm, indeg_hbm, idx_v, hist_v):
    c = lax.axis_index("c")
    s = lax.axis_index("s")

    @pl.when(c == 0)
    def _():
        pltpu.sync_copy(src_hbm.at[s], idx_v)

    @pl.when(c == 1)
    def _():
        pltpu.sync_copy(dst_hbm.at[s], idx_v)

    zeros16 = jnp.zeros((16,), jnp.int32)

    @pl.loop(0, RD * 8)
    def _(i):
        hist_v[pl.ds(i * 16, 16)] = zeros16

    ones16 = jnp.ones((16,), jnp.int32)

    @pl.loop(0, NCH)
    def _(j):
        @pl.loop(0, CH // 16)
        def _(k):
            idx16 = idx_v[j, pl.ds(k * 16, 16)]
            plsc.addupdate_scatter(hist_v, [idx16], ones16)

    @pl.when(c == 0)
    def _():
        pltpu.sync_copy(hist_v, outdeg_hbm.at[s])

    @pl.when(c == 1)
    def _():
        pltpu.sync_copy(hist_v, indeg_hbm.at[s])


def _degsum_body(op_ref, ip_ref, ns_ref, nd_ref):
    od = jnp.sum(op_ref[...], axis=0, keepdims=True).astype(jnp.float32)
    idg = jnp.sum(ip_ref[...], axis=0, keepdims=True).astype(jnp.float32)
    ns_ref[...] = lax.rsqrt(jnp.maximum(od, 1.0))
    nd_ref[...] = lax.rsqrt(jnp.maximum(idg, 1.0))


def _degsum(outp, inp):
    return pl.pallas_call(
        _degsum_body,
        out_shape=[
            jax.ShapeDtypeStruct((1, RD * 128), jnp.float32),
            jax.ShapeDtypeStruct((1, RD * 128), jnp.float32),
        ],
    )(outp, inp)


@functools.partial(
    pl.kernel,
    out_type=(
        jax.ShapeDtypeStruct((NS, RPS, HALF), jnp.float32),
        jax.ShapeDtypeStruct((NS, RPS, HALF), jnp.float32),
    ),
    mesh=_MESH,
    scratch_types=[
        pltpu.VMEM((PCH, CH), jnp.int32),
        pltpu.VMEM((PCH, CH), jnp.int32),
        pltpu.VMEM((CH, HALF), jnp.float32),
        pltpu.VMEM((CH, HALF), jnp.float32),
        pltpu.VMEM_SHARED((NP, HALF), jnp.float32),
        pltpu.SemaphoreType.DMA,
        pltpu.SemaphoreType.DMA,
    ],
)
def _agg_kernel(h0_hbm, h1_hbm, src_hbm, dst_hbm, z_hbm,
                agg0_hbm, agg1_hbm, idxs_v, idxd_v, rows0_v, rows1_v,
                agg_sh, gs0, gs1):
    c = lax.axis_index("c")
    s = lax.axis_index("s")
    pltpu.sync_copy(z_hbm.at[s], agg_sh.at[pl.ds(s * RPS, RPS)])
    plsc.subcore_barrier()

    def run(h_hbm, out_hbm):
        def gather(j, buf, sem):
            pltpu.async_copy(h_hbm.at[idxs_v.at[j]], buf, sem)

        def gwait(j, buf, sem):
            pltpu.make_async_copy(h_hbm.at[idxs_v.at[j]], buf, sem).wait()

        def scatter(j, buf, sem):
            pltpu.async_copy(buf, agg_sh.at[idxd_v.at[j]], sem,
                             add=True).wait()

        for ph in range(PH):
            pltpu.sync_copy(src_hbm.at[s].at[pl.ds(ph * PCH, PCH)], idxs_v)
            pltpu.sync_copy(dst_hbm.at[s].at[pl.ds(ph * PCH, PCH)], idxd_v)
            gather(0, rows0_v, gs0)

            @pl.loop(0, PCH // 2 - 1)
            def _(p):
                a = 1 + 2 * p
                gather(a, rows1_v, gs1)
                gwait(2 * p, rows0_v, gs0)
                scatter(2 * p, rows0_v, gs0)
                gather(a + 1, rows0_v, gs0)
                gwait(a, rows1_v, gs1)
                scatter(a, rows1_v, gs1)

            gather(PCH - 1, rows1_v, gs1)
            gwait(PCH - 2, rows0_v, gs0)
            scatter(PCH - 2, rows0_v, gs0)
            gwait(PCH - 1, rows1_v, gs1)
            scatter(PCH - 1, rows1_v, gs1)

        plsc.subcore_barrier()
        pltpu.sync_copy(agg_sh.at[pl.ds(s * RPS, RPS)], out_hbm.at[s])

    @pl.when(c == 0)
    def _():
        run(h0_hbm, agg0_hbm)

    @pl.when(c == 1)
    def _():
        run(h1_hbm, agg1_hbm)


_BLK = 1000


def _matmul_body(f_ref, w_ref, y_ref):
    y_ref[...] = jnp.dot(f_ref[...], w_ref[...],
                         preferred_element_type=jnp.float32,
                         precision=lax.Precision.HIGHEST)


def _matmul(feats, W):
    return pl.pallas_call(
        _matmul_body,
        grid=(N // _BLK,),
        in_specs=[
            pl.BlockSpec((_BLK, D), lambda i: (i, 0)),
            pl.BlockSpec((D, D), lambda i: (0, 0)),
        ],
        out_specs=pl.BlockSpec((_BLK, D), lambda i: (i, 0)),
        out_shape=jax.ShapeDtypeStruct((N, D), jnp.float32),
    )(feats, W)


def _scale_body(f_ref, d_ref, h0_ref, h1_ref):
    h = f_ref[...] * d_ref[...]
    h0_ref[...] = h[:, :HALF]
    h1_ref[...] = h[:, HALF:]


def _scale(feats, outdegw):
    return pl.pallas_call(
        _scale_body,
        grid=(N // _BLK,),
        in_specs=[
            pl.BlockSpec((_BLK, D), lambda i: (i, 0)),
            pl.BlockSpec((_BLK, 1), lambda i: (i, 0)),
        ],
        out_specs=[
            pl.BlockSpec((_BLK, HALF), lambda i: (i, 0)),
            pl.BlockSpec((_BLK, HALF), lambda i: (i, 0)),
        ],
        out_shape=[
            jax.ShapeDtypeStruct((N, HALF), jnp.float32),
            jax.ShapeDtypeStruct((N, HALF), jnp.float32),
        ],
    )(feats, outdegw)


def _final_body(a0_ref, a1_ref, d_ref, b_ref, o_ref):
    nd = d_ref[...]
    o_ref[:, :HALF] = jnp.maximum(a0_ref[...] * nd + b_ref[:, :HALF], 0.0)
    o_ref[:, HALF:] = jnp.maximum(a1_ref[...] * nd + b_ref[:, HALF:], 0.0)


def _final(agg0, agg1, indegw, b2d):
    return pl.pallas_call(
        _final_body,
        grid=(N // _BLK,),
        in_specs=[
            pl.BlockSpec((_BLK, HALF), lambda i: (i, 0)),
            pl.BlockSpec((_BLK, HALF), lambda i: (i, 0)),
            pl.BlockSpec((_BLK, 1), lambda i: (i, 0)),
            pl.BlockSpec((1, D), lambda i: (0, 0)),
        ],
        out_specs=pl.BlockSpec((_BLK, D), lambda i: (i, 0)),
        out_shape=jax.ShapeDtypeStruct((N, D), jnp.float32),
    )(agg0, agg1, indegw, b2d)


def kernel(feats, edge_index, W, b):
    trash = jnp.broadcast_to(N + jnp.arange(NS, dtype=jnp.int32)[:, None],
                             (NS, PAD))
    zpad = jnp.zeros((NS, PAD), jnp.int32)
    e0 = edge_index[0].reshape(NS, EPS)
    e1 = edge_index[1].reshape(NS, EPS)
    srcd = jnp.concatenate([e0, trash], axis=1).reshape(NS, NCH, CH)
    srca = jnp.concatenate([e0, zpad], axis=1).reshape(NS, NCH, CH)
    dst = jnp.concatenate([e1, trash], axis=1).reshape(NS, NCH, CH)
    y = _matmul(feats, W)
    outp, inp = _deg_kernel(srcd, dst)
    outdegw, indegw = _degsum(outp, inp)
    outdegw = outdegw.reshape(RD * 128)[:N].reshape(N, 1)
    indegw = indegw.reshape(RD * 128)[:N].reshape(N, 1)
    h0, h1 = _scale(y, outdegw)
    h0 = jnp.pad(h0, ((0, NP - N), (0, 0)))
    h1 = jnp.pad(h1, ((0, NP - N), (0, 0)))
    zerosw = jnp.zeros((NS, RPS, HALF), jnp.float32)
    agg0, agg1 = _agg_kernel(h0, h1, srca, dst, zerosw)
    return _final(agg0.reshape(NP, HALF), agg1.reshape(NP, HALF),
                  indegw, b.reshape(1, D))


# speedup vs baseline: 1.5216x; 1.5216x over previous
import functools

import jax
import jax.numpy as jnp
from jax import lax
from jax.experimental import pallas as pl
from jax.experimental.pallas import tpu as pltpu
from jax.experimental.pallas import tpu_sc as plsc

N = 10000
E = 160000
D = 256
HALF = 128
NS = 16
NP = N + NS
CH = 128
EPS = E // NS
NCH = 80
PAD = NCH * CH - EPS
PH = 2
PCH = NCH // PH
RPS = NP // NS
RD = 80
RDS = RD // NS

_MESH = plsc.VectorSubcoreMesh(core_axis_name="c", subcore_axis_name="s")


@functools.partial(
    pl.kernel,
    out_type=(
        jax.ShapeDtypeStruct((NS, RD * 128), jnp.int32),
        jax.ShapeDtypeStruct((NS, RD * 128), jnp.int32),
    ),
    mesh=_MESH,
    scratch_types=[
        pltpu.VMEM((NCH, CH), jnp.int32),
        pltpu.VMEM((RD * 128,), jnp.int32),
    ],
    compiler_params=pltpu.CompilerParams(needs_layout_passes=False),
)
def _deg_kernel(src_hbm, dst_hbm, outdeg_hbm, indeg_hbm, idx_v, hist_v):
    c = lax.axis_index("c")
    s = lax.axis_index("s")

    @pl.when(c == 0)
    def _():
        pltpu.sync_copy(src_hbm.at[s], idx_v)

    @pl.when(c == 1)
    def _():
        pltpu.sync_copy(dst_hbm.at[s], idx_v)

    zeros16 = jnp.zeros((16,), jnp.int32)

    @pl.loop(0, RD * 8)
    def _(i):
        hist_v[pl.ds(i * 16, 16)] = zeros16

    ones16 = jnp.ones((16,), jnp.int32)

    @pl.loop(0, NCH)
    def _(j):
        @pl.loop(0, CH // 16)
        def _(k):
            idx16 = idx_v[j, pl.ds(k * 16, 16)]
            plsc.addupdate_scatter(hist_v, [idx16], ones16)

    @pl.when(c == 0)
    def _():
        pltpu.sync_copy(hist_v, outdeg_hbm.at[s])

    @pl.when(c == 1)
    def _():
        pltpu.sync_copy(hist_v, indeg_hbm.at[s])


def _degsum_body(op_ref, ip_ref, ns_ref, nd_ref):
    od = jnp.sum(op_ref[...], axis=0, keepdims=True).astype(jnp.float32)
    idg = jnp.sum(ip_ref[...], axis=0, keepdims=True).astype(jnp.float32)
    ns_ref[...] = lax.rsqrt(jnp.maximum(od, 1.0))
    nd_ref[...] = lax.rsqrt(jnp.maximum(idg, 1.0))


def _degsum(outp, inp):
    return pl.pallas_call(
        _degsum_body,
        out_shape=[
            jax.ShapeDtypeStruct((1, RD * 128), jnp.float32),
            jax.ShapeDtypeStruct((1, RD * 128), jnp.float32),
        ],
    )(outp, inp)


@functools.partial(
    pl.kernel,
    out_type=(
        jax.ShapeDtypeStruct((NS, RPS, HALF), jnp.float32),
        jax.ShapeDtypeStruct((NS, RPS, HALF), jnp.float32),
    ),
    mesh=_MESH,
    scratch_types=[
        pltpu.VMEM((PCH, CH), jnp.int32),
        pltpu.VMEM((PCH, CH), jnp.int32),
        pltpu.VMEM((CH, HALF), jnp.float32),
        pltpu.VMEM((CH, HALF), jnp.float32),
        pltpu.VMEM_SHARED((NP, HALF), jnp.float32),
        pltpu.SemaphoreType.DMA,
        pltpu.SemaphoreType.DMA,
    ],
)
def _agg_kernel(h0_hbm, h1_hbm, src_hbm, dst_hbm, z_hbm,
                agg0_hbm, agg1_hbm, idxs_v, idxd_v, rows0_v, rows1_v,
                agg_sh, gs0, gs1):
    c = lax.axis_index("c")
    s = lax.axis_index("s")
    pltpu.sync_copy(z_hbm.at[s], agg_sh.at[pl.ds(s * RPS, RPS)])
    plsc.subcore_barrier()

    def run(h_hbm, out_hbm):
        def gather(j, buf, sem):
            pltpu.async_copy(h_hbm.at[idxs_v.at[j]], buf, sem)

        def gwait(j, buf, sem):
            pltpu.make_async_copy(h_hbm.at[idxs_v.at[j]], buf, sem).wait()

        def scatter(j, buf, sem):
            pltpu.async_copy(buf, agg_sh.at[idxd_v.at[j]], sem,
                             add=True).wait()

        for ph in range(PH):
            pltpu.sync_copy(src_hbm.at[s].at[pl.ds(ph * PCH, PCH)], idxs_v)
            pltpu.sync_copy(dst_hbm.at[s].at[pl.ds(ph * PCH, PCH)], idxd_v)
            gather(0, rows0_v, gs0)

            @pl.loop(0, PCH // 2 - 1)
            def _(p):
                a = 1 + 2 * p
                gather(a, rows1_v, gs1)
                gwait(2 * p, rows0_v, gs0)
                scatter(2 * p, rows0_v, gs0)
                gather(a + 1, rows0_v, gs0)
                gwait(a, rows1_v, gs1)
                scatter(a, rows1_v, gs1)

            gather(PCH - 1, rows1_v, gs1)
            gwait(PCH - 2, rows0_v, gs0)
            scatter(PCH - 2, rows0_v, gs0)
            gwait(PCH - 1, rows1_v, gs1)
            scatter(PCH - 1, rows1_v, gs1)

        plsc.subcore_barrier()
        pltpu.sync_copy(agg_sh.at[pl.ds(s * RPS, RPS)], out_hbm.at[s])

    @pl.when(c == 0)
    def _():
        run(h0_hbm, agg0_hbm)

    @pl.when(c == 1)
    def _():
        run(h1_hbm, agg1_hbm)


_BLK = 1000


def _matmul_body(f_ref, w_ref, y_ref):
    y_ref[...] = jnp.dot(f_ref[...], w_ref[...],
                         preferred_element_type=jnp.float32,
                         precision=lax.Precision.HIGHEST)


def _matmul(feats, W):
    return pl.pallas_call(
        _matmul_body,
        grid=(N // _BLK,),
        in_specs=[
            pl.BlockSpec((_BLK, D), lambda i: (i, 0)),
            pl.BlockSpec((D, D), lambda i: (0, 0)),
        ],
        out_specs=pl.BlockSpec((_BLK, D), lambda i: (i, 0)),
        out_shape=jax.ShapeDtypeStruct((N, D), jnp.float32),
    )(feats, W)


def _scale_body(f_ref, d_ref, h0_ref, h1_ref):
    h = f_ref[...] * d_ref[...]
    h0_ref[...] = h[:, :HALF]
    h1_ref[...] = h[:, HALF:]


def _scale(feats, outdegw):
    return pl.pallas_call(
        _scale_body,
        grid=(N // _BLK,),
        in_specs=[
            pl.BlockSpec((_BLK, D), lambda i: (i, 0)),
            pl.BlockSpec((_BLK, 1), lambda i: (i, 0)),
        ],
        out_specs=[
            pl.BlockSpec((_BLK, HALF), lambda i: (i, 0)),
            pl.BlockSpec((_BLK, HALF), lambda i: (i, 0)),
        ],
        out_shape=[
            jax.ShapeDtypeStruct((N, HALF), jnp.float32),
            jax.ShapeDtypeStruct((N, HALF), jnp.float32),
        ],
    )(feats, outdegw)


def _final_body(a0_ref, a1_ref, d_ref, b_ref, o_ref):
    nd = d_ref[...]
    o_ref[:, :HALF] = jnp.maximum(a0_ref[...] * nd + b_ref[:, :HALF], 0.0)
    o_ref[:, HALF:] = jnp.maximum(a1_ref[...] * nd + b_ref[:, HALF:], 0.0)


def _final(agg0, agg1, indegw, b2d):
    return pl.pallas_call(
        _final_body,
        grid=(N // _BLK,),
        in_specs=[
            pl.BlockSpec((_BLK, HALF), lambda i: (i, 0)),
            pl.BlockSpec((_BLK, HALF), lambda i: (i, 0)),
            pl.BlockSpec((_BLK, 1), lambda i: (i, 0)),
            pl.BlockSpec((1, D), lambda i: (0, 0)),
        ],
        out_specs=pl.BlockSpec((_BLK, D), lambda i: (i, 0)),
        out_shape=jax.ShapeDtypeStruct((N, D), jnp.float32),
    )(agg0, agg1, indegw, b2d)


def kernel(feats, edge_index, W, b):
    trash = jnp.broadcast_to(N + jnp.arange(NS, dtype=jnp.int32)[:, None],
                             (NS, PAD))
    zpad = jnp.broadcast_to(jnp.arange(NS, dtype=jnp.int32)[:, None],
                            (NS, PAD))
    e0 = edge_index[0].reshape(NS, EPS)
    e1 = edge_index[1].reshape(NS, EPS)
    srcd = jnp.concatenate([e0, trash], axis=1).reshape(NS, NCH, CH)
    srca = jnp.concatenate([e0, zpad], axis=1).reshape(NS, NCH, CH)
    dst = jnp.concatenate([e1, trash], axis=1).reshape(NS, NCH, CH)
    y = _matmul(feats, W)
    outp, inp = _deg_kernel(srcd, dst)
    outdegw, indegw = _degsum(outp, inp)
    outdegw = outdegw.reshape(RD * 128)[:N].reshape(N, 1)
    indegw = indegw.reshape(RD * 128)[:N].reshape(N, 1)
    h0, h1 = _scale(y, outdegw)
    h0 = jnp.pad(h0, ((0, NP - N), (0, 0)))
    h1 = jnp.pad(h1, ((0, NP - N), (0, 0)))
    zerosw = jnp.zeros((NS, RPS, HALF), jnp.float32)
    agg0, agg1 = _agg_kernel(h0, h1, srca, dst, zerosw)
    return _final(agg0.reshape(NP, HALF), agg1.reshape(NP, HALF),
                  indegw, b.reshape(1, D))


# speedup vs baseline: 1.5723x; 1.0333x over previous
import functools

import jax
import jax.numpy as jnp
from jax import lax
from jax.experimental import pallas as pl
from jax.experimental.pallas import tpu as pltpu
from jax.experimental.pallas import tpu_sc as plsc

N = 10000
E = 160000
D = 256
HALF = 128
NS = 16
NP = N + NS
CH = 128
EPS = E // NS
NCH = 80
PAD = NCH * CH - EPS
PH = 2
PCH = NCH // PH
RPS = NP // NS
RD = 80
RDS = RD // NS

_MESH = plsc.VectorSubcoreMesh(core_axis_name="c", subcore_axis_name="s")


@functools.partial(
    pl.kernel,
    out_type=(
        jax.ShapeDtypeStruct((NS, RD * 128), jnp.int32),
        jax.ShapeDtypeStruct((NS, RD * 128), jnp.int32),
    ),
    mesh=_MESH,
    scratch_types=[
        pltpu.VMEM((NCH, CH), jnp.int32),
        pltpu.VMEM((RD * 128,), jnp.int32),
    ],
    compiler_params=pltpu.CompilerParams(needs_layout_passes=False),
)
def _deg_kernel(src_hbm, dst_hbm, outdeg_hbm, indeg_hbm, idx_v, hist_v):
    c = lax.axis_index("c")
    s = lax.axis_index("s")

    @pl.when(c == 0)
    def _():
        pltpu.sync_copy(src_hbm.at[s], idx_v)

    @pl.when(c == 1)
    def _():
        pltpu.sync_copy(dst_hbm.at[s], idx_v)

    zeros16 = jnp.zeros((16,), jnp.int32)

    @pl.loop(0, RD * 8)
    def _(i):
        hist_v[pl.ds(i * 16, 16)] = zeros16

    ones16 = jnp.ones((16,), jnp.int32)

    @pl.loop(0, NCH)
    def _(j):
        @pl.loop(0, CH // 16)
        def _(k):
            idx16 = idx_v[j, pl.ds(k * 16, 16)]
            plsc.addupdate_scatter(hist_v, [idx16], ones16)

    @pl.when(c == 0)
    def _():
        pltpu.sync_copy(hist_v, outdeg_hbm.at[s])

    @pl.when(c == 1)
    def _():
        pltpu.sync_copy(hist_v, indeg_hbm.at[s])


def _degsum_body(op_ref, ip_ref, ns_ref, nd_ref):
    od = jnp.sum(op_ref[...], axis=0, keepdims=True).astype(jnp.float32)
    idg = jnp.sum(ip_ref[...], axis=0, keepdims=True).astype(jnp.float32)
    ns_ref[...] = lax.rsqrt(jnp.maximum(od, 1.0))
    nd_ref[...] = lax.rsqrt(jnp.maximum(idg, 1.0))


def _degsum(outp, inp):
    return pl.pallas_call(
        _degsum_body,
        out_shape=[
            jax.ShapeDtypeStruct((1, RD * 128), jnp.float32),
            jax.ShapeDtypeStruct((1, RD * 128), jnp.float32),
        ],
    )(outp, inp)


@functools.partial(
    pl.kernel,
    out_type=(
        jax.ShapeDtypeStruct((NS, RPS, HALF), jnp.float32),
        jax.ShapeDtypeStruct((NS, RPS, HALF), jnp.float32),
    ),
    mesh=_MESH,
    scratch_types=[
        pltpu.VMEM((PCH, CH), jnp.int32),
        pltpu.VMEM((PCH, CH), jnp.int32),
        pltpu.VMEM((CH, HALF), jnp.float32),
        pltpu.VMEM((CH, HALF), jnp.float32),
        pltpu.VMEM_SHARED((NP, HALF), jnp.float32),
        pltpu.SemaphoreType.DMA,
        pltpu.SemaphoreType.DMA,
    ],
)
def _agg_kernel(h0_hbm, h1_hbm, src_hbm, dst_hbm, z_hbm,
                agg0_hbm, agg1_hbm, idxs_v, idxd_v, rows0_v, rows1_v,
                agg_sh, gs0, gs1):
    c = lax.axis_index("c")
    s = lax.axis_index("s")
    pltpu.sync_copy(z_hbm.at[s], agg_sh.at[pl.ds(s * RPS, RPS)])
    plsc.subcore_barrier()

    def run(h_hbm, out_hbm):
        def gather(j, buf, sem):
            pltpu.async_copy(h_hbm.at[idxs_v.at[j]], buf, sem)

        def gwait(j, buf, sem):
            pltpu.make_async_copy(h_hbm.at[idxs_v.at[j]], buf, sem).wait()

        def scatter(j, buf, sem):
            pltpu.async_copy(buf, agg_sh.at[idxd_v.at[j]], sem,
                             add=True).wait()

        for ph in range(PH):
            pltpu.sync_copy(src_hbm.at[s].at[pl.ds(ph * PCH, PCH)], idxs_v)
            pltpu.sync_copy(dst_hbm.at[s].at[pl.ds(ph * PCH, PCH)], idxd_v)
            gather(0, rows0_v, gs0)

            @pl.loop(0, PCH // 2 - 1)
            def _(p):
                a = 1 + 2 * p
                gather(a, rows1_v, gs1)
                gwait(2 * p, rows0_v, gs0)
                scatter(2 * p, rows0_v, gs0)
                gather(a + 1, rows0_v, gs0)
                gwait(a, rows1_v, gs1)
                scatter(a, rows1_v, gs1)

            gather(PCH - 1, rows1_v, gs1)
            gwait(PCH - 2, rows0_v, gs0)
            scatter(PCH - 2, rows0_v, gs0)
            gwait(PCH - 1, rows1_v, gs1)
            scatter(PCH - 1, rows1_v, gs1)

        plsc.subcore_barrier()
        pltpu.sync_copy(agg_sh.at[pl.ds(s * RPS, RPS)], out_hbm.at[s])

    @pl.when(c == 0)
    def _():
        run(h0_hbm, agg0_hbm)

    @pl.when(c == 1)
    def _():
        run(h1_hbm, agg1_hbm)


_BLK = 1000


def _matmul_body(f_ref, w_ref, y_ref):
    y_ref[...] = jnp.dot(f_ref[...], w_ref[...],
                         preferred_element_type=jnp.float32,
                         precision=lax.Precision.HIGHEST)


def _matmul(feats, W):
    return pl.pallas_call(
        _matmul_body,
        grid=(N // _BLK,),
        in_specs=[
            pl.BlockSpec((_BLK, D), lambda i: (i, 0)),
            pl.BlockSpec((D, D), lambda i: (0, 0)),
        ],
        out_specs=pl.BlockSpec((_BLK, D), lambda i: (i, 0)),
        out_shape=jax.ShapeDtypeStruct((N, D), jnp.float32),
    )(feats, W)


def _scale_body(f_ref, d_ref, h0_ref, h1_ref):
    h = f_ref[...] * d_ref[...]
    h0_ref[...] = h[:, :HALF]
    h1_ref[...] = h[:, HALF:]


def _scale(feats, outdegw):
    return pl.pallas_call(
        _scale_body,
        grid=(N // _BLK,),
        in_specs=[
            pl.BlockSpec((_BLK, D), lambda i: (i, 0)),
            pl.BlockSpec((_BLK, 1), lambda i: (i, 0)),
        ],
        out_specs=[
            pl.BlockSpec((_BLK, HALF), lambda i: (i, 0)),
            pl.BlockSpec((_BLK, HALF), lambda i: (i, 0)),
        ],
        out_shape=[
            jax.ShapeDtypeStruct((N, HALF), jnp.float32),
            jax.ShapeDtypeStruct((N, HALF), jnp.float32),
        ],
    )(feats, outdegw)


def _final_body(a0_ref, a1_ref, d_ref, b_ref, o_ref):
    nd = d_ref[...]
    o_ref[:, :HALF] = jnp.maximum(a0_ref[...] * nd + b_ref[:, :HALF], 0.0)
    o_ref[:, HALF:] = jnp.maximum(a1_ref[...] * nd + b_ref[:, HALF:], 0.0)


def _final(agg0, agg1, indegw, b2d):
    return pl.pallas_call(
        _final_body,
        grid=(N // _BLK,),
        in_specs=[
            pl.BlockSpec((_BLK, HALF), lambda i: (i, 0)),
            pl.BlockSpec((_BLK, HALF), lambda i: (i, 0)),
            pl.BlockSpec((_BLK, 1), lambda i: (i, 0)),
            pl.BlockSpec((1, D), lambda i: (0, 0)),
        ],
        out_specs=pl.BlockSpec((_BLK, D), lambda i: (i, 0)),
        out_shape=jax.ShapeDtypeStruct((N, D), jnp.float32),
    )(agg0, agg1, indegw, b2d)


def kernel(feats, edge_index, W, b):
    trash = jnp.broadcast_to(N + jnp.arange(NS, dtype=jnp.int32)[:, None],
                             (NS, PAD))
    zpad = jnp.broadcast_to(jnp.arange(NS, dtype=jnp.int32)[:, None],
                            (NS, PAD))
    e0 = edge_index[0].reshape(NS, EPS)
    e1 = edge_index[1].reshape(NS, EPS)
    srcd = jnp.concatenate([e0, trash], axis=1).reshape(NS, NCH, CH)
    srca = jnp.concatenate([e0, zpad], axis=1).reshape(NS, NCH, CH)
    dst = jnp.concatenate([e1, trash], axis=1).reshape(NS, NCH, CH)
    y = _matmul(feats, W)
    outp, inp = _deg_kernel(srcd, dst)
    outdegw, indegw = _degsum(outp, inp)
    outdegw = outdegw.reshape(RD * 128)[:N].reshape(N, 1)
    indegw = indegw.reshape(RD * 128)[:N].reshape(N, 1)
    h0, h1 = _scale(y, outdegw)
    zerosw = jnp.zeros((NS, RPS, HALF), jnp.float32)
    agg0, agg1 = _agg_kernel(h0, h1, srca, dst, zerosw)
    return _final(agg0.reshape(NP, HALF), agg1.reshape(NP, HALF),
                  indegw, b.reshape(1, D))
